# Initial kernel scaffold; baseline (speedup 1.0000x reference)
import jax
import jax.numpy as jnp
from jax.experimental import pallas as pl


def kernel(x):
    raise NotImplementedError("write your pallas kernel here")


# speedup vs baseline: 19.8349x; 19.8349x over previous
import functools
import math

import jax
import jax.numpy as jnp
from jax.experimental import pallas as pl

_SPARSITY = 0.05


def _kwta_body(k_active, x_ref, out_ref):
    x = x_ref[...]
    n = x.shape[1]

    i = jax.lax.bitcast_convert_type(x, jnp.int32)
    keys = jnp.where(i < 0, i ^ jnp.int32(0x7FFFFFFF), i)

    int_min = jnp.int32(-2147483648)
    v0 = jnp.full((x.shape[0], 1), int_min, dtype=jnp.int32)
    c0 = jnp.full((x.shape[0], 1), n, dtype=jnp.int32)

    def step(it, carry):
        v, cnt_v = carry
        bit = jax.lax.shift_left(jnp.int32(1), jnp.int32(31) - it.astype(jnp.int32))
        trial = v + bit
        cnt = jnp.sum((keys >= trial).astype(jnp.int32), axis=1, keepdims=True)
        take = cnt >= k_active
        return jnp.where(take, trial, v), jnp.where(take, cnt, cnt_v)

    v, cnt_v = jax.lax.fori_loop(0, 32, step, (v0, c0))

    vnext = jnp.max(jnp.where(keys < v, keys, int_min), axis=1, keepdims=True)
    vk1 = jnp.where(cnt_v >= k_active + 1, v, vnext)

    def to_f32(s):
        return jax.lax.bitcast_convert_type(
            jnp.where(s < 0, s ^ jnp.int32(0x7FFFFFFF), s), jnp.float32
        )

    thr = (to_f32(v) + to_f32(vk1)) * 0.5
    out_ref[...] = (x > thr).astype(jnp.float32)


def kernel(x):
    batch, emb = x.shape
    k_active = math.ceil(_SPARSITY * emb)
    return pl.pallas_call(
        functools.partial(_kwta_body, k_active),
        out_shape=jax.ShapeDtypeStruct((batch, emb), jnp.float32),
    )(x)
